# baseline (speedup 1.0000x reference)
import functools

import jax
import jax.numpy as jnp
from jax import lax
from jax.experimental import pallas as pl
from jax.experimental.pallas import tpu as pltpu

E = 8
N = 576
P = 16
D = 256
IN = 3 * P * P
MID = 64 * P * P
LAT = 64
HW = P // 4
OUT = LAT * HW * HW
NHEADS = 8

TILE = 64
TMAX = 16
MCH = 2048
MST = MID // MCH
EPAD = 128


def _router_kernel(feat_ref, w1_ref, b1_ref, w2_ref, b2_ref, eid_ref):
    h = jnp.maximum(
        lax.dot_general(feat_ref[...], w1_ref[...], (((1,), (1,)), ((), ())),
                        preferred_element_type=jnp.float32) + b1_ref[...],
        0.0)
    logits = lax.dot_general(h, w2_ref[...], (((1,), (1,)), ((), ())),
                             preferred_element_type=jnp.float32) + b2_ref[...]
    mx = jnp.max(logits, axis=1, keepdims=True)
    lane = lax.broadcasted_iota(jnp.int32, (N, EPAD), 1)
    cand = jnp.where(logits >= mx, lane, EPAD - 1)
    eid_ref[...] = jnp.min(cand, axis=1, keepdims=True)


def _moe_kernel(tile_e_ref, valid_ref, tok_ref,
                feat_ref, wp_ref, bp_ref, wv_ref, bv_ref, wo_ref, bo_ref,
                lng_ref, lnb_ref, w1_ref, b1_ref, w2_ref, b2_ref,
                out_ref,
                xg_ref, emb_ref, acc_ref):
    m = pl.program_id(0)
    t = pl.program_id(1)
    e = tile_e_ref[t]

    @pl.when(valid_ref[t] == 1)
    def _run():
        @pl.when(m == 0)
        def _embed():
            def gather_row(j, _):
                xg_ref[pl.ds(j, 1), :] = feat_ref[pl.ds(tok_ref[t, j], 1), :]
                return 0
            lax.fori_loop(0, TILE, gather_row, 0, unroll=8)
            xg = xg_ref[...]
            emb = lax.dot_general(xg, wp_ref[e], (((1,), (1,)), ((), ())),
                                  preferred_element_type=jnp.float32)
            emb = emb + bp_ref[pl.ds(e, 1), :]
            v = lax.dot_general(emb, wv_ref[e], (((1,), (1,)), ((), ())),
                                preferred_element_type=jnp.float32)
            v = v + bv_ref[pl.ds(e, 1), :]
            attn = lax.dot_general(v, wo_ref[e], (((1,), (1,)), ((), ())),
                                   preferred_element_type=jnp.float32)
            y = emb + attn + bo_ref[pl.ds(e, 1), :]
            mu = jnp.mean(y, axis=1, keepdims=True)
            yc = y - mu
            var = jnp.mean(yc * yc, axis=1, keepdims=True)
            emb_ref[t] = (yc * lax.rsqrt(var + 1e-5) * lng_ref[pl.ds(e, 1), :]
                          + lnb_ref[pl.ds(e, 1), :])

        emb = emb_ref[t].astype(jnp.bfloat16)
        hp = jnp.maximum(
            lax.dot_general(emb, w1_ref[0].astype(jnp.bfloat16),
                            (((1,), (1,)), ((), ())),
                            preferred_element_type=jnp.float32) + b1_ref[0, 0],
            0.0)
        contrib = lax.dot_general(hp.astype(jnp.bfloat16),
                                  w2_ref[0].astype(jnp.bfloat16),
                                  (((1,), (1,)), ((), ())),
                                  preferred_element_type=jnp.float32)

        @pl.when(m == 0)
        def _init():
            acc_ref[t] = contrib

        @pl.when(m > 0)
        def _acc():
            acc_ref[t] = acc_ref[t] + contrib

        @pl.when(m == MST - 1)
        def _finish():
            acc_ref[t] = jnp.tanh(acc_ref[t] + b2_ref[pl.ds(e, 1), :])

            def scatter_row(j, _):
                out_ref[pl.ds(tok_ref[t, j], 1), :] = acc_ref[t, pl.ds(j, 1), :]
                return 0
            lax.fori_loop(0, TILE, scatter_row, 0, unroll=8)


@jax.jit
def kernel(x, Wr1, br1, Wr2, br2, Wp, bp, Wqkv, bqkv, Wo, bo, ln_g, ln_b,
           W1, b1, W2, b2):
    feat = x.reshape(N, IN)

    Wr2p = jnp.zeros((EPAD, 256), jnp.float32).at[:E].set(Wr2)
    br2p = jnp.full((1, EPAD), -1e30, jnp.float32).at[0, :E].set(br2)
    eid2 = pl.pallas_call(
        _router_kernel,
        out_shape=jax.ShapeDtypeStruct((N, 1), jnp.int32),
    )(feat, Wr1, br1.reshape(1, 256), Wr2p, br2p)
    eid = eid2[:, 0]

    sort_idx = jnp.argsort(eid, stable=True).astype(jnp.int32)
    counts = jnp.sum(jax.nn.one_hot(eid, E, dtype=jnp.int32), axis=0)
    offsets = jnp.concatenate([jnp.zeros((1,), jnp.int32),
                               jnp.cumsum(counts)[:-1]])
    ntiles = (counts + TILE - 1) // TILE
    tile_csum = jnp.cumsum(ntiles)
    total_tiles = tile_csum[-1]
    tfirst = tile_csum - ntiles
    tt = jnp.arange(TMAX, dtype=jnp.int32)
    e_of_t = jnp.searchsorted(tile_csum, tt, side="right").astype(jnp.int32)
    valid = (tt < total_tiles).astype(jnp.int32)
    last_e = jnp.searchsorted(tile_csum, total_tiles - 1,
                              side="right").astype(jnp.int32)
    tile_e = jnp.where(valid == 1, e_of_t, last_e)
    start = offsets[tile_e] + (tt - tfirst[tile_e]) * TILE
    s = start[:, None] + jnp.arange(TILE, dtype=jnp.int32)[None, :]
    s_end = offsets[tile_e] + counts[tile_e] - 1
    s = jnp.minimum(s, s_end[:, None])
    s = jnp.clip(s, 0, N - 1)
    tok = sort_idx[s]

    Wv = Wqkv[:, 2 * D:, :]
    bv = bqkv[:, 2 * D:]

    grid_spec = pltpu.PrefetchScalarGridSpec(
        num_scalar_prefetch=3,
        grid=(MST, TMAX),
        in_specs=[
            pl.BlockSpec((N, IN), lambda m, t, te, va, tk: (0, 0)),
            pl.BlockSpec((E, D, IN), lambda m, t, te, va, tk: (0, 0, 0)),
            pl.BlockSpec((E, D), lambda m, t, te, va, tk: (0, 0)),
            pl.BlockSpec((E, D, D), lambda m, t, te, va, tk: (0, 0, 0)),
            pl.BlockSpec((E, D), lambda m, t, te, va, tk: (0, 0)),
            pl.BlockSpec((E, D, D), lambda m, t, te, va, tk: (0, 0, 0)),
            pl.BlockSpec((E, D), lambda m, t, te, va, tk: (0, 0)),
            pl.BlockSpec((E, D), lambda m, t, te, va, tk: (0, 0)),
            pl.BlockSpec((E, D), lambda m, t, te, va, tk: (0, 0)),
            pl.BlockSpec((1, MCH, D), lambda m, t, te, va, tk: (te[t], m, 0)),
            pl.BlockSpec((1, 1, 1, MCH), lambda m, t, te, va, tk: (te[t], m, 0, 0)),
            pl.BlockSpec((1, OUT, MCH), lambda m, t, te, va, tk: (te[t], 0, m)),
            pl.BlockSpec((E, OUT), lambda m, t, te, va, tk: (0, 0)),
        ],
        out_specs=pl.BlockSpec((N, OUT), lambda m, t, te, va, tk: (0, 0)),
        scratch_shapes=[
            pltpu.VMEM((TILE, IN), jnp.float32),
            pltpu.VMEM((TMAX, TILE, D), jnp.float32),
            pltpu.VMEM((TMAX, TILE, OUT), jnp.float32),
        ],
    )

    out = pl.pallas_call(
        _moe_kernel,
        grid_spec=grid_spec,
        out_shape=jax.ShapeDtypeStruct((N, OUT), jnp.float32),
        compiler_params=pltpu.CompilerParams(
            dimension_semantics=("arbitrary", "arbitrary")),
    )(tile_e, valid, tok,
      feat, Wp, bp, Wv, bv, Wo, bo, ln_g, ln_b, W1,
      b1.reshape(E, MST, 1, MCH), W2, b2)

    return out.reshape(N, LAT, HW, HW)


# speedup vs baseline: 1.4274x; 1.4274x over previous
import functools

import jax
import jax.numpy as jnp
from jax import lax
from jax.experimental import pallas as pl
from jax.experimental.pallas import tpu as pltpu

E = 8
N = 576
P = 16
D = 256
IN = 3 * P * P
MID = 64 * P * P
LAT = 64
HW = P // 4
OUT = LAT * HW * HW
NHEADS = 8

TILE = 128
TMAX = 12
MCH = 2048
MST = MID // MCH
EPAD = 128


def _router_kernel(feat_ref, w1_ref, b1_ref, w2_ref, b2_ref, eid_ref):
    h = jnp.maximum(
        lax.dot_general(feat_ref[...], w1_ref[...], (((1,), (1,)), ((), ())),
                        preferred_element_type=jnp.float32) + b1_ref[...],
        0.0)
    logits = lax.dot_general(h, w2_ref[...], (((1,), (1,)), ((), ())),
                             preferred_element_type=jnp.float32) + b2_ref[...]
    mx = jnp.max(logits, axis=1, keepdims=True)
    lane = lax.broadcasted_iota(jnp.int32, (N, EPAD), 1)
    cand = jnp.where(logits >= mx, lane, EPAD - 1)
    eid_ref[...] = jnp.min(cand, axis=1, keepdims=True)


def _moe_kernel(tile_e_ref, valid_ref, tok_ref,
                feat_ref, wp_ref, bp_ref, wv_ref, bv_ref, wo_ref, bo_ref,
                lng_ref, lnb_ref, w1_ref, b1_ref, w2_ref, b2_ref,
                out_ref,
                xg_ref, emb_ref, acc_ref):
    t = pl.program_id(0)
    m = pl.program_id(1)
    e = tile_e_ref[t]

    @pl.when(valid_ref[t] == 1)
    def _run():
        @pl.when(m == 0)
        def _embed():
            def gather_row(j, _):
                xg_ref[pl.ds(j, 1), :] = feat_ref[pl.ds(tok_ref[t, j], 1), :]
                return 0
            lax.fori_loop(0, TILE, gather_row, 0, unroll=8)
            xg = xg_ref[...]
            emb = lax.dot_general(xg, wp_ref[e], (((1,), (1,)), ((), ())),
                                  preferred_element_type=jnp.float32)
            emb = emb + bp_ref[pl.ds(e, 1), :]
            v = lax.dot_general(emb, wv_ref[e], (((1,), (1,)), ((), ())),
                                preferred_element_type=jnp.float32)
            v = v + bv_ref[pl.ds(e, 1), :]
            attn = lax.dot_general(v, wo_ref[e], (((1,), (1,)), ((), ())),
                                   preferred_element_type=jnp.float32)
            y = emb + attn + bo_ref[pl.ds(e, 1), :]
            mu = jnp.mean(y, axis=1, keepdims=True)
            yc = y - mu
            var = jnp.mean(yc * yc, axis=1, keepdims=True)
            emb_ref[...] = (yc * lax.rsqrt(var + 1e-5) * lng_ref[pl.ds(e, 1), :]
                          + lnb_ref[pl.ds(e, 1), :])

        emb = emb_ref[...].astype(jnp.bfloat16)
        hp = jnp.maximum(
            lax.dot_general(emb, w1_ref[0].astype(jnp.bfloat16),
                            (((1,), (1,)), ((), ())),
                            preferred_element_type=jnp.float32) + b1_ref[0, 0],
            0.0)
        contrib = lax.dot_general(hp.astype(jnp.bfloat16),
                                  w2_ref[0].astype(jnp.bfloat16),
                                  (((1,), (1,)), ((), ())),
                                  preferred_element_type=jnp.float32)

        @pl.when(m == 0)
        def _init():
            acc_ref[...] = contrib

        @pl.when(m > 0)
        def _acc():
            acc_ref[...] = acc_ref[...] + contrib

        @pl.when(m == MST - 1)
        def _finish():
            acc_ref[...] = jnp.tanh(acc_ref[...] + b2_ref[pl.ds(e, 1), :])

            def scatter_row(j, _):
                out_ref[pl.ds(tok_ref[t, j], 1), :] = acc_ref[pl.ds(j, 1), :]
                return 0
            lax.fori_loop(0, TILE, scatter_row, 0, unroll=8)


@jax.jit
def kernel(x, Wr1, br1, Wr2, br2, Wp, bp, Wqkv, bqkv, Wo, bo, ln_g, ln_b,
           W1, b1, W2, b2):
    feat = x.reshape(N, IN)

    Wr2p = jnp.zeros((EPAD, 256), jnp.float32).at[:E].set(Wr2)
    br2p = jnp.full((1, EPAD), -1e30, jnp.float32).at[0, :E].set(br2)
    eid2 = pl.pallas_call(
        _router_kernel,
        out_shape=jax.ShapeDtypeStruct((N, 1), jnp.int32),
    )(feat, Wr1, br1.reshape(1, 256), Wr2p, br2p)
    eid = eid2[:, 0]

    sort_idx = jnp.argsort(eid, stable=True).astype(jnp.int32)
    counts = jnp.sum(jax.nn.one_hot(eid, E, dtype=jnp.int32), axis=0)
    offsets = jnp.concatenate([jnp.zeros((1,), jnp.int32),
                               jnp.cumsum(counts)[:-1]])
    ntiles = (counts + TILE - 1) // TILE
    tile_csum = jnp.cumsum(ntiles)
    total_tiles = tile_csum[-1]
    tfirst = tile_csum - ntiles
    tt = jnp.arange(TMAX, dtype=jnp.int32)
    e_of_t = jnp.searchsorted(tile_csum, tt, side="right").astype(jnp.int32)
    valid = (tt < total_tiles).astype(jnp.int32)
    last_e = jnp.searchsorted(tile_csum, total_tiles - 1,
                              side="right").astype(jnp.int32)
    tile_e = jnp.where(valid == 1, e_of_t, last_e)
    start = offsets[tile_e] + (tt - tfirst[tile_e]) * TILE
    s = start[:, None] + jnp.arange(TILE, dtype=jnp.int32)[None, :]
    s_end = offsets[tile_e] + counts[tile_e] - 1
    s = jnp.minimum(s, s_end[:, None])
    s = jnp.clip(s, 0, N - 1)
    tok = sort_idx[s]

    Wv = Wqkv[:, 2 * D:, :]
    bv = bqkv[:, 2 * D:]

    def _mm(m, va, t):
        return jnp.where(va[t] == 1, m, MST - 1)

    grid_spec = pltpu.PrefetchScalarGridSpec(
        num_scalar_prefetch=3,
        grid=(TMAX, MST),
        in_specs=[
            pl.BlockSpec((N, IN), lambda t, m, te, va, tk: (0, 0)),
            pl.BlockSpec((E, D, IN), lambda t, m, te, va, tk: (0, 0, 0)),
            pl.BlockSpec((E, D), lambda t, m, te, va, tk: (0, 0)),
            pl.BlockSpec((E, D, D), lambda t, m, te, va, tk: (0, 0, 0)),
            pl.BlockSpec((E, D), lambda t, m, te, va, tk: (0, 0)),
            pl.BlockSpec((E, D, D), lambda t, m, te, va, tk: (0, 0, 0)),
            pl.BlockSpec((E, D), lambda t, m, te, va, tk: (0, 0)),
            pl.BlockSpec((E, D), lambda t, m, te, va, tk: (0, 0)),
            pl.BlockSpec((E, D), lambda t, m, te, va, tk: (0, 0)),
            pl.BlockSpec((1, MCH, D),
                         lambda t, m, te, va, tk: (te[t], _mm(m, va, t), 0)),
            pl.BlockSpec((1, 1, 1, MCH),
                         lambda t, m, te, va, tk: (te[t], _mm(m, va, t), 0, 0)),
            pl.BlockSpec((1, OUT, MCH),
                         lambda t, m, te, va, tk: (te[t], 0, _mm(m, va, t))),
            pl.BlockSpec((E, OUT), lambda t, m, te, va, tk: (0, 0)),
        ],
        out_specs=pl.BlockSpec((N, OUT), lambda t, m, te, va, tk: (0, 0)),
        scratch_shapes=[
            pltpu.VMEM((TILE, IN), jnp.float32),
            pltpu.VMEM((TILE, D), jnp.float32),
            pltpu.VMEM((TILE, OUT), jnp.float32),
        ],
    )

    out = pl.pallas_call(
        _moe_kernel,
        grid_spec=grid_spec,
        out_shape=jax.ShapeDtypeStruct((N, OUT), jnp.float32),
        compiler_params=pltpu.CompilerParams(
            dimension_semantics=("arbitrary", "arbitrary")),
    )(tile_e, valid, tok,
      feat, Wp, bp, Wv, bv, Wo, bo, ln_g, ln_b, W1,
      b1.reshape(E, MST, 1, MCH), W2, b2)

    return out.reshape(N, LAT, HW, HW)


# speedup vs baseline: 1.4724x; 1.0315x over previous
import functools

import jax
import jax.numpy as jnp
from jax import lax
from jax.experimental import pallas as pl
from jax.experimental.pallas import tpu as pltpu

E = 8
N = 576
P = 16
D = 256
IN = 3 * P * P
MID = 64 * P * P
LAT = 64
HW = P // 4
OUT = LAT * HW * HW
NHEADS = 8

TILE = 128
TMAX = 12
MCH = 4096
MST = MID // MCH
EPAD = 128


def _router_kernel(feat_ref, w1_ref, b1_ref, w2_ref, b2_ref, eid_ref):
    h = jnp.maximum(
        lax.dot_general(feat_ref[...], w1_ref[...], (((1,), (1,)), ((), ())),
                        preferred_element_type=jnp.float32) + b1_ref[...],
        0.0)
    logits = lax.dot_general(h, w2_ref[...], (((1,), (1,)), ((), ())),
                             preferred_element_type=jnp.float32) + b2_ref[...]
    mx = jnp.max(logits, axis=1, keepdims=True)
    lane = lax.broadcasted_iota(jnp.int32, (N, EPAD), 1)
    cand = jnp.where(logits >= mx, lane, EPAD - 1)
    eid_ref[...] = jnp.min(cand, axis=1, keepdims=True)


def _moe_kernel(tile_e_ref, valid_ref, tok_ref,
                feat_ref, wp_ref, bp_ref, wv_ref, bv_ref, wo_ref, bo_ref,
                lng_ref, lnb_ref, w1_ref, b1_ref, w2_ref, b2_ref,
                out_ref,
                xg_ref, emb_ref, acc_ref):
    t = pl.program_id(0)
    m = pl.program_id(1)
    e = tile_e_ref[t]

    @pl.when(valid_ref[t] == 1)
    def _run():
        @pl.when(m == 0)
        def _embed():
            def gather_row(j, _):
                xg_ref[pl.ds(j, 1), :] = feat_ref[pl.ds(tok_ref[t, j], 1), :]
                return 0
            lax.fori_loop(0, TILE, gather_row, 0, unroll=8)
            xg = xg_ref[...]
            emb = lax.dot_general(xg, wp_ref[e], (((1,), (1,)), ((), ())),
                                  preferred_element_type=jnp.float32)
            emb = emb + bp_ref[pl.ds(e, 1), :]
            v = lax.dot_general(emb, wv_ref[e], (((1,), (1,)), ((), ())),
                                preferred_element_type=jnp.float32)
            v = v + bv_ref[pl.ds(e, 1), :]
            attn = lax.dot_general(v, wo_ref[e], (((1,), (1,)), ((), ())),
                                   preferred_element_type=jnp.float32)
            y = emb + attn + bo_ref[pl.ds(e, 1), :]
            mu = jnp.mean(y, axis=1, keepdims=True)
            yc = y - mu
            var = jnp.mean(yc * yc, axis=1, keepdims=True)
            emb_ref[...] = (yc * lax.rsqrt(var + 1e-5) * lng_ref[pl.ds(e, 1), :]
                          + lnb_ref[pl.ds(e, 1), :])

        emb = emb_ref[...].astype(jnp.bfloat16)
        hp = jnp.maximum(
            lax.dot_general(emb, w1_ref[0].astype(jnp.bfloat16),
                            (((1,), (1,)), ((), ())),
                            preferred_element_type=jnp.float32) + b1_ref[0, 0],
            0.0)
        contrib = lax.dot_general(hp.astype(jnp.bfloat16),
                                  w2_ref[0].astype(jnp.bfloat16),
                                  (((1,), (1,)), ((), ())),
                                  preferred_element_type=jnp.float32)

        @pl.when(m == 0)
        def _init():
            acc_ref[...] = contrib

        @pl.when(m > 0)
        def _acc():
            acc_ref[...] = acc_ref[...] + contrib

        @pl.when(m == MST - 1)
        def _finish():
            acc_ref[...] = jnp.tanh(acc_ref[...] + b2_ref[pl.ds(e, 1), :])

            def scatter_row(j, _):
                out_ref[pl.ds(tok_ref[t, j], 1), :] = acc_ref[pl.ds(j, 1), :]
                return 0
            lax.fori_loop(0, TILE, scatter_row, 0, unroll=8)


@jax.jit
def kernel(x, Wr1, br1, Wr2, br2, Wp, bp, Wqkv, bqkv, Wo, bo, ln_g, ln_b,
           W1, b1, W2, b2):
    feat = x.reshape(N, IN)

    Wr2p = jnp.zeros((EPAD, 256), jnp.float32).at[:E].set(Wr2)
    br2p = jnp.full((1, EPAD), -1e30, jnp.float32).at[0, :E].set(br2)
    eid2 = pl.pallas_call(
        _router_kernel,
        out_shape=jax.ShapeDtypeStruct((N, 1), jnp.int32),
    )(feat, Wr1, br1.reshape(1, 256), Wr2p, br2p)
    eid = eid2[:, 0]

    sort_idx = jnp.argsort(eid, stable=True).astype(jnp.int32)
    counts = jnp.sum(jax.nn.one_hot(eid, E, dtype=jnp.int32), axis=0)
    offsets = jnp.concatenate([jnp.zeros((1,), jnp.int32),
                               jnp.cumsum(counts)[:-1]])
    ntiles = (counts + TILE - 1) // TILE
    tile_csum = jnp.cumsum(ntiles)
    total_tiles = tile_csum[-1]
    tfirst = tile_csum - ntiles
    tt = jnp.arange(TMAX, dtype=jnp.int32)
    e_of_t = jnp.searchsorted(tile_csum, tt, side="right").astype(jnp.int32)
    valid = (tt < total_tiles).astype(jnp.int32)
    last_e = jnp.searchsorted(tile_csum, total_tiles - 1,
                              side="right").astype(jnp.int32)
    tile_e = jnp.where(valid == 1, e_of_t, last_e)
    start = offsets[tile_e] + (tt - tfirst[tile_e]) * TILE
    s = start[:, None] + jnp.arange(TILE, dtype=jnp.int32)[None, :]
    s_end = offsets[tile_e] + counts[tile_e] - 1
    s = jnp.minimum(s, s_end[:, None])
    s = jnp.clip(s, 0, N - 1)
    tok = sort_idx[s]

    Wv = Wqkv[:, 2 * D:, :]
    bv = bqkv[:, 2 * D:]

    def _mm(m, va, t):
        return jnp.where(va[t] == 1, m, MST - 1)

    grid_spec = pltpu.PrefetchScalarGridSpec(
        num_scalar_prefetch=3,
        grid=(TMAX, MST),
        in_specs=[
            pl.BlockSpec((N, IN), lambda t, m, te, va, tk: (0, 0)),
            pl.BlockSpec((E, D, IN), lambda t, m, te, va, tk: (0, 0, 0)),
            pl.BlockSpec((E, D), lambda t, m, te, va, tk: (0, 0)),
            pl.BlockSpec((E, D, D), lambda t, m, te, va, tk: (0, 0, 0)),
            pl.BlockSpec((E, D), lambda t, m, te, va, tk: (0, 0)),
            pl.BlockSpec((E, D, D), lambda t, m, te, va, tk: (0, 0, 0)),
            pl.BlockSpec((E, D), lambda t, m, te, va, tk: (0, 0)),
            pl.BlockSpec((E, D), lambda t, m, te, va, tk: (0, 0)),
            pl.BlockSpec((E, D), lambda t, m, te, va, tk: (0, 0)),
            pl.BlockSpec((1, MCH, D),
                         lambda t, m, te, va, tk: (te[t], _mm(m, va, t), 0)),
            pl.BlockSpec((1, 1, 1, MCH),
                         lambda t, m, te, va, tk: (te[t], _mm(m, va, t), 0, 0)),
            pl.BlockSpec((1, OUT, MCH),
                         lambda t, m, te, va, tk: (te[t], 0, _mm(m, va, t))),
            pl.BlockSpec((E, OUT), lambda t, m, te, va, tk: (0, 0)),
        ],
        out_specs=pl.BlockSpec((N, OUT), lambda t, m, te, va, tk: (0, 0)),
        scratch_shapes=[
            pltpu.VMEM((TILE, IN), jnp.float32),
            pltpu.VMEM((TILE, D), jnp.float32),
            pltpu.VMEM((TILE, OUT), jnp.float32),
        ],
    )

    out = pl.pallas_call(
        _moe_kernel,
        grid_spec=grid_spec,
        out_shape=jax.ShapeDtypeStruct((N, OUT), jnp.float32),
        compiler_params=pltpu.CompilerParams(
            dimension_semantics=("arbitrary", "arbitrary")),
    )(tile_e, valid, tok,
      feat, Wp, bp, Wv, bv, Wo, bo, ln_g, ln_b, W1,
      b1.reshape(E, MST, 1, MCH), W2, b2)

    return out.reshape(N, LAT, HW, HW)


# speedup vs baseline: 1.5072x; 1.0237x over previous
import functools

import jax
import jax.numpy as jnp
from jax import lax
from jax.experimental import pallas as pl
from jax.experimental.pallas import tpu as pltpu
from jax.experimental.pallas import tpu_sc as plsc

E = 8
N = 576
P = 16
D = 256
IN = 3 * P * P
MID = 64 * P * P
LAT = 64
HW = P // 4
OUT = LAT * HW * HW
NHEADS = 8

TILE = 128
TMAX = 12
MCH = 4096
MST = MID // MCH
EPAD = 128


def _router_kernel(feat_ref, w1_ref, b1_ref, w2_ref, b2_ref, eid_ref):
    h = jnp.maximum(
        lax.dot_general(feat_ref[...], w1_ref[...], (((1,), (1,)), ((), ())),
                        preferred_element_type=jnp.float32) + b1_ref[...],
        0.0)
    logits = lax.dot_general(h, w2_ref[...], (((1,), (1,)), ((), ())),
                             preferred_element_type=jnp.float32) + b2_ref[...]
    mx = jnp.max(logits, axis=1, keepdims=True)
    lane = lax.broadcasted_iota(jnp.int32, (N, EPAD), 1)
    cand = jnp.where(logits >= mx, lane, EPAD - 1)
    eid_ref[...] = jnp.min(cand, axis=1, keepdims=True)


def _meta_kernel(eid_hbm, meta_hbm, srt_hbm,
                 eid_s, srt_s, meta_s):
    CNT, OFF, CUR, TF, TF2, TE, VA, TS, TN = 0, 8, 16, 24, 32, 40, 56, 72, 88
    cid = lax.axis_index("c")

    @pl.when(cid == 0)
    def _run():
        pltpu.sync_copy(eid_hbm, eid_s)
        for e in range(E):
            meta_s[CNT + e] = 0

        def c_body(i, carry):
            ei = eid_s[i]
            meta_s[CNT + ei] = meta_s[CNT + ei] + 1
            return carry
        lax.fori_loop(0, N, c_body, 0)

        off_acc = jnp.int32(0)
        tf_acc = jnp.int32(0)
        for e in range(E):
            c = meta_s[CNT + e]
            nt = (c + TILE - 1) // TILE
            meta_s[OFF + e] = off_acc
            meta_s[CUR + e] = off_acc
            meta_s[TF + e] = tf_acc
            meta_s[TF2 + e] = tf_acc + nt
            off_acc = off_acc + c
            tf_acc = tf_acc + nt
        total_tiles = tf_acc

        def s_body(i, carry):
            ei = eid_s[i]
            p = meta_s[CUR + ei]
            srt_s[p] = i
            meta_s[CUR + ei] = p + 1
            return carry
        lax.fori_loop(0, N, s_body, 0)

        tl = total_tiles - 1
        e_last = jnp.int32(0)
        for e in range(E):
            e_last = e_last + jnp.where(tl >= meta_s[TF2 + e], 1, 0)
        for t in range(TMAX):
            e_sel = jnp.int32(0)
            for e in range(E):
                e_sel = e_sel + jnp.where(t >= meta_s[TF2 + e], 1, 0)
            vld = t < total_tiles
            e_t = jnp.where(vld, e_sel, e_last)
            meta_s[TE + t] = e_t
            meta_s[VA + t] = jnp.where(vld, 1, 0)
            meta_s[TS + t] = meta_s[OFF + e_t] + (t - meta_s[TF + e_t]) * TILE
            meta_s[TN + t] = meta_s[OFF + e_t] + meta_s[CNT + e_t] - 1
        for t in range(TMAX, 16):
            meta_s[TE + t] = 0
            meta_s[VA + t] = 0
            meta_s[TS + t] = 0
            meta_s[TN + t] = 0

        pltpu.sync_copy(srt_s, srt_hbm)
        pltpu.sync_copy(meta_s, meta_hbm)


_meta = functools.partial(
    pl.kernel,
    mesh=plsc.ScalarSubcoreMesh(axis_name="c", num_cores=2),
    out_type=[
        jax.ShapeDtypeStruct((128,), jnp.int32),
        jax.ShapeDtypeStruct((640,), jnp.int32),
    ],
    scratch_types=[
        pltpu.SMEM((640,), jnp.int32),
        pltpu.SMEM((640,), jnp.int32),
        pltpu.SMEM((128,), jnp.int32),
    ],
)(_meta_kernel)


def _moe_kernel(te_ref, va_ref, ts_ref, tn_ref, srt_ref,
                feat_ref, wp_ref, bp_ref, wv_ref, bv_ref, wo_ref, bo_ref,
                lng_ref, lnb_ref, w1_ref, b1_ref, w2_ref, b2_ref,
                out_ref,
                xg_ref, emb_ref, acc_ref):
    t = pl.program_id(0)
    m = pl.program_id(1)
    e = te_ref[t]

    @pl.when(va_ref[t] == 1)
    def _run():
        @pl.when(m == 0)
        def _embed():
            def gather_row(j, _):
                s = jnp.minimum(ts_ref[t] + j, tn_ref[t])
                xg_ref[pl.ds(j, 1), :] = feat_ref[pl.ds(srt_ref[s], 1), :]
                return 0
            lax.fori_loop(0, TILE, gather_row, 0, unroll=8)
            xg = xg_ref[...]
            emb = lax.dot_general(xg, wp_ref[e], (((1,), (1,)), ((), ())),
                                  preferred_element_type=jnp.float32)
            emb = emb + bp_ref[pl.ds(e, 1), :]
            v = lax.dot_general(emb, wv_ref[e], (((1,), (1,)), ((), ())),
                                preferred_element_type=jnp.float32)
            v = v + bv_ref[pl.ds(e, 1), :]
            attn = lax.dot_general(v, wo_ref[e], (((1,), (1,)), ((), ())),
                                   preferred_element_type=jnp.float32)
            y = emb + attn + bo_ref[pl.ds(e, 1), :]
            mu = jnp.mean(y, axis=1, keepdims=True)
            yc = y - mu
            var = jnp.mean(yc * yc, axis=1, keepdims=True)
            emb_ref[...] = (yc * lax.rsqrt(var + 1e-5)
                            * lng_ref[pl.ds(e, 1), :]
                            + lnb_ref[pl.ds(e, 1), :])

        emb = emb_ref[...].astype(jnp.bfloat16)
        hp = jnp.maximum(
            lax.dot_general(emb, w1_ref[0].astype(jnp.bfloat16),
                            (((1,), (1,)), ((), ())),
                            preferred_element_type=jnp.float32) + b1_ref[0, 0],
            0.0)
        contrib = lax.dot_general(hp.astype(jnp.bfloat16),
                                  w2_ref[0].astype(jnp.bfloat16),
                                  (((1,), (1,)), ((), ())),
                                  preferred_element_type=jnp.float32)

        @pl.when(m == 0)
        def _init():
            acc_ref[...] = contrib

        @pl.when(m > 0)
        def _acc():
            acc_ref[...] = acc_ref[...] + contrib

        @pl.when(m == MST - 1)
        def _finish():
            acc_ref[...] = jnp.tanh(acc_ref[...] + b2_ref[pl.ds(e, 1), :])

            def scatter_row(j, _):
                s = jnp.minimum(ts_ref[t] + j, tn_ref[t])
                out_ref[pl.ds(srt_ref[s], 1), :] = acc_ref[pl.ds(j, 1), :]
                return 0
            lax.fori_loop(0, TILE, scatter_row, 0, unroll=8)


@jax.jit
def kernel(x, Wr1, br1, Wr2, br2, Wp, bp, Wqkv, bqkv, Wo, bo, ln_g, ln_b,
           W1, b1, W2, b2):
    feat = x.reshape(N, IN)

    Wr2p = jnp.zeros((EPAD, 256), jnp.float32).at[:E].set(Wr2)
    br2p = jnp.full((1, EPAD), -1e30, jnp.float32).at[0, :E].set(br2)
    eid2 = pl.pallas_call(
        _router_kernel,
        out_shape=jax.ShapeDtypeStruct((N, 1), jnp.int32),
    )(feat, Wr1, br1.reshape(1, 256), Wr2p, br2p)
    eid = eid2[:, 0]

    eid640 = jnp.zeros((640,), jnp.int32).at[:N].set(eid)
    meta128, srt640 = _meta(eid640)
    tile_e = meta128[40:40 + TMAX]
    valid = meta128[56:56 + TMAX]
    tstart = meta128[72:72 + TMAX]
    tend = meta128[88:88 + TMAX]
    srt = srt640[:N]

    Wv = Wqkv[:, 2 * D:, :]
    bv = bqkv[:, 2 * D:]

    def _mm(m, va, t):
        return jnp.where(va[t] == 1, m, MST - 1)

    grid_spec = pltpu.PrefetchScalarGridSpec(
        num_scalar_prefetch=5,
        grid=(TMAX, MST),
        in_specs=[
            pl.BlockSpec((N, IN), lambda t, m, te, va, ts, tn, sr: (0, 0)),
            pl.BlockSpec((E, D, IN),
                         lambda t, m, te, va, ts, tn, sr: (0, 0, 0)),
            pl.BlockSpec((E, D), lambda t, m, te, va, ts, tn, sr: (0, 0)),
            pl.BlockSpec((E, D, D),
                         lambda t, m, te, va, ts, tn, sr: (0, 0, 0)),
            pl.BlockSpec((E, D), lambda t, m, te, va, ts, tn, sr: (0, 0)),
            pl.BlockSpec((E, D, D),
                         lambda t, m, te, va, ts, tn, sr: (0, 0, 0)),
            pl.BlockSpec((E, D), lambda t, m, te, va, ts, tn, sr: (0, 0)),
            pl.BlockSpec((E, D), lambda t, m, te, va, ts, tn, sr: (0, 0)),
            pl.BlockSpec((E, D), lambda t, m, te, va, ts, tn, sr: (0, 0)),
            pl.BlockSpec((1, MCH, D),
                         lambda t, m, te, va, ts, tn, sr:
                         (te[t], _mm(m, va, t), 0)),
            pl.BlockSpec((1, 1, 1, MCH),
                         lambda t, m, te, va, ts, tn, sr:
                         (te[t], _mm(m, va, t), 0, 0)),
            pl.BlockSpec((1, OUT, MCH),
                         lambda t, m, te, va, ts, tn, sr:
                         (te[t], 0, _mm(m, va, t))),
            pl.BlockSpec((E, OUT), lambda t, m, te, va, ts, tn, sr: (0, 0)),
        ],
        out_specs=pl.BlockSpec((N, OUT),
                               lambda t, m, te, va, ts, tn, sr: (0, 0)),
        scratch_shapes=[
            pltpu.VMEM((TILE, IN), jnp.float32),
            pltpu.VMEM((TILE, D), jnp.float32),
            pltpu.VMEM((TILE, OUT), jnp.float32),
        ],
    )

    out = pl.pallas_call(
        _moe_kernel,
        grid_spec=grid_spec,
        out_shape=jax.ShapeDtypeStruct((N, OUT), jnp.float32),
        compiler_params=pltpu.CompilerParams(
            dimension_semantics=("arbitrary", "arbitrary")),
    )(tile_e, valid, tstart, tend, srt,
      feat, Wp, bp, Wv, bv, Wo, bo, ln_g, ln_b, W1,
      b1.reshape(E, MST, 1, MCH), W2, b2)

    return out.reshape(N, LAT, HW, HW)
